# baseline (speedup 1.0000x reference)
import functools

import jax
import jax.numpy as jnp
from jax import lax
from jax.experimental import pallas as pl
from jax.experimental.pallas import tpu as pltpu
from jax.experimental.pallas import tpu_sc as plsc

N_NODES = 10000
N_EDGES = 320000
D = 128

NC = 2
NS = 16
NW = NC * NS

N_PAD = 10240
E_PAD = 327680
EB = E_PAD // NW
CHUNK = 128
NCHUNK = EB // CHUNK
HALF = NCHUNK // 2
ROWS_PER_TILE = N_PAD // NS
NBUF = 2


def _deg_body(dst_hbm, deg_out, dst_v, ones_v, zero_v, deg_sh):
    c = lax.axis_index("c")
    s = lax.axis_index("s")
    w = c * NS + s

    pltpu.sync_copy(dst_hbm.at[w], dst_v)

    z = jnp.zeros((16,), jnp.float32)
    for j in range(8):
        ones_v[pl.ds(j * 16, 16)] = z + 1.0
    for j in range(ROWS_PER_TILE // 16):
        zero_v[pl.ds(j * 16, 16)] = z
    pltpu.sync_copy(zero_v, deg_sh.at[pl.ds(s * ROWS_PER_TILE, ROWS_PER_TILE)])
    plsc.subcore_barrier()

    def chunk(i, carry):
        pltpu.sync_copy(ones_v, deg_sh.at[dst_v.at[i]], add=True)
        return carry

    lax.fori_loop(0, NCHUNK, chunk, 0)
    plsc.subcore_barrier()

    pltpu.sync_copy(
        deg_sh.at[pl.ds(s * ROWS_PER_TILE, ROWS_PER_TILE)],
        deg_out.at[c, pl.ds(s * ROWS_PER_TILE, ROWS_PER_TILE)],
    )


def _deg_partials(dst_r):
    mesh = plsc.VectorSubcoreMesh(
        core_axis_name="c", subcore_axis_name="s", num_cores=NC, num_subcores=NS
    )
    f = pl.kernel(
        _deg_body,
        out_type=jax.ShapeDtypeStruct((NC, N_PAD), jnp.float32),
        mesh=mesh,
        scratch_types=[
            pltpu.VMEM((NCHUNK, CHUNK), jnp.int32),
            pltpu.VMEM((CHUNK,), jnp.float32),
            pltpu.VMEM((ROWS_PER_TILE,), jnp.float32),
            pltpu.VMEM_SHARED((N_PAD,), jnp.float32),
        ],
    )
    return f(dst_r)


def _msg_body(h2_hbm, src_hbm, dst_hbm, acc_out,
              src_v, dst_v, rows0, rows1, acc_sh, gsem):
    c = lax.axis_index("c")
    s = lax.axis_index("s")
    w = c * NS + s
    bufs = (rows0, rows1)

    pltpu.sync_copy(src_hbm.at[w], src_v)

    row0 = s * ROWS_PER_TILE

    @pl.when(c == 0)
    def _():
        pltpu.sync_copy(
            h2_hbm.at[pl.ds(row0, ROWS_PER_TILE)],
            acc_sh.at[pl.ds(row0, ROWS_PER_TILE)],
        )

    @pl.when(c == 1)
    def _():
        z = jnp.zeros((16,), jnp.float32)

        def zrow(i, carry):
            for j in range(D // 16):
                rows0[i, pl.ds(j * 16, 16)] = z
            return carry

        lax.fori_loop(0, CHUNK, zrow, 0)
        for r in range(ROWS_PER_TILE // CHUNK):
            pltpu.sync_copy(rows0, acc_sh.at[pl.ds(row0 + r * CHUNK, CHUNK)])

    plsc.subcore_barrier()

    J = 8

    for h in range(2):
        pltpu.sync_copy(dst_hbm.at[w, pl.ds(h * HALF, HALF)], dst_v)
        hbase = h * HALF

        def outer(k, carry):
            base = hbase + k * J
            lbase = k * J
            g_prev = pltpu.async_copy(
                h2_hbm.at[src_v.at[base]], bufs[0], gsem.at[0]
            )
            for j in range(J):
                b = j % 2
                nb = (j + 1) % 2
                if j + 1 < J:
                    g_next = pltpu.async_copy(
                        h2_hbm.at[src_v.at[base + j + 1]], bufs[nb], gsem.at[nb]
                    )
                g_prev.wait()
                pltpu.sync_copy(bufs[b], acc_sh.at[dst_v.at[lbase + j]], add=True)
                if j + 1 < J:
                    g_prev = g_next
            return carry

        lax.fori_loop(0, HALF // J, outer, 0)
    plsc.subcore_barrier()

    pltpu.sync_copy(
        acc_sh.at[pl.ds(s * ROWS_PER_TILE, ROWS_PER_TILE)],
        acc_out.at[c, pl.ds(s * ROWS_PER_TILE, ROWS_PER_TILE)],
    )


def _msg_partials(h2, src_m, dst_m):
    mesh = plsc.VectorSubcoreMesh(
        core_axis_name="c", subcore_axis_name="s", num_cores=NC, num_subcores=NS
    )
    f = pl.kernel(
        _msg_body,
        out_type=jax.ShapeDtypeStruct((NC, N_PAD, D), jnp.float32),
        mesh=mesh,
        scratch_types=[
            pltpu.VMEM((NCHUNK, CHUNK), jnp.int32),
            pltpu.VMEM((HALF, CHUNK), jnp.int32),
            pltpu.VMEM((CHUNK, D), jnp.float32),
            pltpu.VMEM((CHUNK, D), jnp.float32),
            pltpu.VMEM_SHARED((N_PAD, D), jnp.float32),
            pltpu.SemaphoreType.DMA((NBUF,)),
        ],
    )
    return f(h2, src_m, dst_m)


def _h2_body(emb_ref, w_ref, deg_ref, out_ref):
    deg = deg_ref[0] + deg_ref[1] + 1.0
    dis = lax.rsqrt(deg)
    h = jnp.dot(emb_ref[...], w_ref[...], preferred_element_type=jnp.float32)
    out_ref[...] = h * dis


def _h2(emb_pad, W, deg):
    bm = 1024
    grid = N_PAD // bm
    return pl.pallas_call(
        _h2_body,
        grid=(grid,),
        in_specs=[
            pl.BlockSpec((bm, D), lambda i: (i, 0)),
            pl.BlockSpec((D, D), lambda i: (0, 0)),
            pl.BlockSpec((NC, bm, 1), lambda i: (0, i, 0)),
        ],
        out_specs=pl.BlockSpec((bm, D), lambda i: (i, 0)),
        out_shape=jax.ShapeDtypeStruct((N_PAD, D), jnp.float32),
    )(emb_pad, W, deg)


def _final_body(acc_ref, deg_ref, b_ref, out_ref):
    deg = deg_ref[0] + deg_ref[1] + 1.0
    dis = lax.rsqrt(deg)
    tot = acc_ref[0] + acc_ref[1]
    out_ref[...] = tot * dis + b_ref[...]


def _final(acc, deg, b2d):
    bm = 1024
    grid = N_PAD // bm
    return pl.pallas_call(
        _final_body,
        grid=(grid,),
        in_specs=[
            pl.BlockSpec((NC, bm, D), lambda i: (0, i, 0)),
            pl.BlockSpec((NC, bm, 1), lambda i: (0, i, 0)),
            pl.BlockSpec((1, D), lambda i: (0, 0)),
        ],
        out_specs=pl.BlockSpec((bm, D), lambda i: (i, 0)),
        out_shape=jax.ShapeDtypeStruct((N_PAD, D), jnp.float32),
    )(acc, deg, b2d)


@jax.jit
def kernel(embedding, top_to_bottom_edge_index, W, b):
    src = top_to_bottom_edge_index[0].astype(jnp.int32)
    dst = top_to_bottom_edge_index[1].astype(jnp.int32)
    pad = E_PAD - N_EDGES
    pad_ar = jnp.arange(pad, dtype=jnp.int32)
    src_pad = pad_ar % N_NODES
    dst_pad = N_NODES + pad_ar % (N_PAD - N_NODES)
    src_m = jnp.concatenate([src, src_pad]).reshape(NW, NCHUNK, CHUNK)
    dst_m = jnp.concatenate([dst, dst_pad]).reshape(NW, NCHUNK, CHUNK)
    dst_r = dst_m
    emb_pad = jnp.pad(embedding, ((0, N_PAD - N_NODES), (0, 0)))

    deg = _deg_partials(dst_r).reshape(NC, N_PAD, 1)
    h2 = _h2(emb_pad, W, deg)
    acc = _msg_partials(h2, src_m, dst_m)
    out = _final(acc, deg, b.reshape(1, D))
    return out[:N_NODES]


# speedup vs baseline: 1.1704x; 1.1704x over previous
import jax
import jax.numpy as jnp
from jax import lax
from jax.experimental import pallas as pl
from jax.experimental.pallas import tpu as pltpu
from jax.experimental.pallas import tpu_sc as plsc

N_NODES = 10000
N_EDGES = 320000
D = 128

NC = 2
NS = 16
NW = NC * NS

EB = N_EDGES // NW
CHUNK = 125
NCHUNK = EB // CHUNK
HALF = NCHUNK // 2
J = 8
DEG_PAD = NC * NS * 320
ACC_PAD = 10240
ROWS_PER_TILE = ACC_PAD // NS


def _deg_body(edge_hbm, deg_out, dst_v, ones_v, zero_v, deg_sh):
    c = lax.axis_index("c")
    s = lax.axis_index("s")
    w = c * NS + s

    pltpu.sync_copy(edge_hbm.at[1, w], dst_v)

    z = jnp.zeros((16,), jnp.float32)
    for j in range(8):
        ones_v[pl.ds(j * 16, 16)] = z + 1.0
    for j in range(640 // 16):
        zero_v[pl.ds(j * 16, 16)] = z
    pltpu.sync_copy(zero_v, deg_sh.at[pl.ds(s * 640, 640)])
    plsc.subcore_barrier()

    def chunk(i, carry):
        pltpu.sync_copy(
            ones_v.at[pl.ds(0, CHUNK)], deg_sh.at[dst_v.at[i]], add=True
        )
        return carry

    lax.fori_loop(0, NCHUNK, chunk, 0)
    plsc.subcore_barrier()

    pltpu.sync_copy(
        deg_sh.at[pl.ds(s * 640, 640)],
        deg_out.at[c, pl.ds(s * 640, 640)],
    )


def _deg_partials(edge_r):
    mesh = plsc.VectorSubcoreMesh(
        core_axis_name="c", subcore_axis_name="s", num_cores=NC, num_subcores=NS
    )
    f = pl.kernel(
        _deg_body,
        out_type=jax.ShapeDtypeStruct((NC, DEG_PAD), jnp.float32),
        mesh=mesh,
        scratch_types=[
            pltpu.VMEM((NCHUNK, CHUNK), jnp.int32),
            pltpu.VMEM((128,), jnp.float32),
            pltpu.VMEM((640,), jnp.float32),
            pltpu.VMEM_SHARED((DEG_PAD,), jnp.float32),
        ],
    )
    return f(edge_r)


def _msg_body(h2_hbm, edge_hbm, acc_out, src_v, dst_v, rows0, rows1,
              acc_sh, gsem):
    c = lax.axis_index("c")
    s = lax.axis_index("s")
    w = c * NS + s
    bufs = (rows0, rows1)
    row0 = s * ROWS_PER_TILE

    pltpu.sync_copy(edge_hbm.at[0, w], src_v)

    z = jnp.zeros((16,), jnp.float32)

    def zrow(i, carry):
        for j in range(D // 16):
            rows0[i, pl.ds(j * 16, 16)] = z
        return carry

    lax.fori_loop(0, 128, zrow, 0)
    for r in range(ROWS_PER_TILE // 128):
        pltpu.sync_copy(
            rows0.at[pl.ds(0, 128)],
            acc_sh.at[pl.ds(row0 + r * 128, 128)],
        )

    plsc.subcore_barrier()

    for h in range(2):
        pltpu.sync_copy(edge_hbm.at[1, w, pl.ds(h * HALF, HALF)], dst_v)
        hbase = h * HALF

        def outer(k, carry):
            base = hbase + k * J
            lbase = k * J
            g_prev = pltpu.async_copy(
                h2_hbm.at[src_v.at[base]], bufs[0], gsem.at[0]
            )
            for j in range(J):
                b = j % 2
                nb = (j + 1) % 2
                if j + 1 < J:
                    g_next = pltpu.async_copy(
                        h2_hbm.at[src_v.at[base + j + 1]], bufs[nb],
                        gsem.at[nb],
                    )
                g_prev.wait()
                pltpu.sync_copy(
                    bufs[b], acc_sh.at[dst_v.at[lbase + j]], add=True
                )
                if j + 1 < J:
                    g_prev = g_next
            return carry

        lax.fori_loop(0, HALF // J, outer, 0)
    plsc.subcore_barrier()

    pltpu.sync_copy(
        acc_sh.at[pl.ds(row0, ROWS_PER_TILE)],
        acc_out.at[c, pl.ds(row0, ROWS_PER_TILE)],
    )


def _msg_partials(h2, edge_r):
    mesh = plsc.VectorSubcoreMesh(
        core_axis_name="c", subcore_axis_name="s", num_cores=NC, num_subcores=NS
    )
    f = pl.kernel(
        _msg_body,
        out_type=jax.ShapeDtypeStruct((NC, ACC_PAD, D), jnp.float32),
        mesh=mesh,
        scratch_types=[
            pltpu.VMEM((NCHUNK, CHUNK), jnp.int32),
            pltpu.VMEM((HALF, CHUNK), jnp.int32),
            pltpu.VMEM((CHUNK, D), jnp.float32),
            pltpu.VMEM((CHUNK, D), jnp.float32),
            pltpu.VMEM_SHARED((ACC_PAD, D), jnp.float32),
            pltpu.SemaphoreType.DMA((2,)),
        ],
    )
    return f(h2, edge_r)


def _h2_body(emb_ref, w_ref, deg_ref, out_ref, dis_ref):
    deg = deg_ref[0, :N_NODES] + deg_ref[1, :N_NODES] + 1.0
    dis = lax.rsqrt(deg)[:, None]
    h = jnp.dot(emb_ref[...], w_ref[...], preferred_element_type=jnp.float32)
    out_ref[...] = h * dis
    dis_ref[...] = dis


def _h2(embedding, W, deg):
    return pl.pallas_call(
        _h2_body,
        out_shape=[
            jax.ShapeDtypeStruct((N_NODES, D), jnp.float32),
            jax.ShapeDtypeStruct((N_NODES, 1), jnp.float32),
        ],
    )(embedding, W, deg)


def _final_body(acc_ref, h2_ref, dis_ref, b_ref, out_ref):
    tot = acc_ref[0] + acc_ref[1] + h2_ref[...]
    out_ref[...] = tot * dis_ref[...] + b_ref[...]


def _final(acc, h2, dis, b2d):
    bm = 2000
    grid = N_NODES // bm
    return pl.pallas_call(
        _final_body,
        grid=(grid,),
        in_specs=[
            pl.BlockSpec((NC, bm, D), lambda i: (0, i, 0)),
            pl.BlockSpec((bm, D), lambda i: (i, 0)),
            pl.BlockSpec((bm, 1), lambda i: (i, 0)),
            pl.BlockSpec((1, D), lambda i: (0, 0)),
        ],
        out_specs=pl.BlockSpec((bm, D), lambda i: (i, 0)),
        out_shape=jax.ShapeDtypeStruct((N_NODES, D), jnp.float32),
    )(acc, h2, dis, b2d)


@jax.jit
def kernel(embedding, top_to_bottom_edge_index, W, b):
    edge_r = top_to_bottom_edge_index.astype(jnp.int32).reshape(
        2, NW, NCHUNK, CHUNK
    )
    deg = _deg_partials(edge_r)
    h2, dis = _h2(embedding, W, deg)
    acc = _msg_partials(h2, edge_r)
    return _final(acc, h2, dis, b.reshape(1, D))


# speedup vs baseline: 1.1798x; 1.0080x over previous
import jax
import jax.numpy as jnp
from jax import lax
from jax.experimental import pallas as pl
from jax.experimental.pallas import tpu as pltpu
from jax.experimental.pallas import tpu_sc as plsc

N_NODES = 10000
N_EDGES = 320000
D = 128

NC = 2
NS = 16
NW = NC * NS

EB = N_EDGES // NW
CHUNK = 125
NCHUNK = EB // CHUNK
HALF = NCHUNK // 2
J = 10
DEG_PAD = NC * NS * 320
ACC_PAD = 10240
ROWS_PER_TILE = ACC_PAD // NS


def _deg_body(edge_hbm, deg_out, dst_v, ones_v, zero_v, deg_sh):
    c = lax.axis_index("c")
    s = lax.axis_index("s")
    w = c * NS + s

    pltpu.sync_copy(edge_hbm.at[1, w], dst_v)

    z = jnp.zeros((16,), jnp.float32)
    for j in range(8):
        ones_v[pl.ds(j * 16, 16)] = z + 1.0
    for j in range(640 // 16):
        zero_v[pl.ds(j * 16, 16)] = z
    pltpu.sync_copy(zero_v, deg_sh.at[pl.ds(s * 640, 640)])
    plsc.subcore_barrier()

    def chunk(i, carry):
        pltpu.sync_copy(
            ones_v.at[pl.ds(0, CHUNK)], deg_sh.at[dst_v.at[i]], add=True
        )
        return carry

    lax.fori_loop(0, NCHUNK, chunk, 0)
    plsc.subcore_barrier()

    pltpu.sync_copy(
        deg_sh.at[pl.ds(s * 640, 640)],
        deg_out.at[c, pl.ds(s * 640, 640)],
    )


def _deg_partials(edge_r):
    mesh = plsc.VectorSubcoreMesh(
        core_axis_name="c", subcore_axis_name="s", num_cores=NC, num_subcores=NS
    )
    f = pl.kernel(
        _deg_body,
        out_type=jax.ShapeDtypeStruct((NC, DEG_PAD), jnp.float32),
        mesh=mesh,
        scratch_types=[
            pltpu.VMEM((NCHUNK, CHUNK), jnp.int32),
            pltpu.VMEM((128,), jnp.float32),
            pltpu.VMEM((640,), jnp.float32),
            pltpu.VMEM_SHARED((DEG_PAD,), jnp.float32),
        ],
    )
    return f(edge_r)


def _msg_body(h2_hbm, edge_hbm, acc_out, src_v, dst_v, rows0, rows1,
              acc_sh, gsem):
    c = lax.axis_index("c")
    s = lax.axis_index("s")
    w = c * NS + s
    bufs = (rows0, rows1)
    row0 = s * ROWS_PER_TILE

    pltpu.sync_copy(edge_hbm.at[0, w], src_v)

    z = jnp.zeros((16,), jnp.float32)

    def zrow(i, carry):
        for j in range(D // 16):
            rows0[i, pl.ds(j * 16, 16)] = z
        return carry

    lax.fori_loop(0, 128, zrow, 0)
    for r in range(ROWS_PER_TILE // 128):
        pltpu.sync_copy(
            rows0.at[pl.ds(0, 128)],
            acc_sh.at[pl.ds(row0 + r * 128, 128)],
        )

    plsc.subcore_barrier()

    for h in range(2):
        pltpu.sync_copy(edge_hbm.at[1, w, pl.ds(h * HALF, HALF)], dst_v)
        hbase = h * HALF

        def outer(k, carry):
            base = hbase + k * J
            lbase = k * J
            g_prev = pltpu.async_copy(
                h2_hbm.at[src_v.at[base]], bufs[0], gsem.at[0]
            )
            for j in range(J):
                b = j % 2
                nb = (j + 1) % 2
                if j + 1 < J:
                    g_next = pltpu.async_copy(
                        h2_hbm.at[src_v.at[base + j + 1]], bufs[nb],
                        gsem.at[nb],
                    )
                g_prev.wait()
                pltpu.sync_copy(
                    bufs[b], acc_sh.at[dst_v.at[lbase + j]], add=True
                )
                if j + 1 < J:
                    g_prev = g_next
            return carry

        lax.fori_loop(0, HALF // J, outer, 0)
    plsc.subcore_barrier()

    pltpu.sync_copy(
        acc_sh.at[pl.ds(row0, ROWS_PER_TILE)],
        acc_out.at[c, pl.ds(row0, ROWS_PER_TILE)],
    )


def _msg_partials(h2, edge_r):
    mesh = plsc.VectorSubcoreMesh(
        core_axis_name="c", subcore_axis_name="s", num_cores=NC, num_subcores=NS
    )
    f = pl.kernel(
        _msg_body,
        out_type=jax.ShapeDtypeStruct((NC, ACC_PAD, D), jnp.float32),
        mesh=mesh,
        scratch_types=[
            pltpu.VMEM((NCHUNK, CHUNK), jnp.int32),
            pltpu.VMEM((HALF, CHUNK), jnp.int32),
            pltpu.VMEM((CHUNK, D), jnp.float32),
            pltpu.VMEM((CHUNK, D), jnp.float32),
            pltpu.VMEM_SHARED((ACC_PAD, D), jnp.float32),
            pltpu.SemaphoreType.DMA((2,)),
        ],
    )
    return f(h2, edge_r)


def _mm_body(emb_ref, w_ref, out_ref):
    out_ref[...] = jnp.dot(
        emb_ref[...], w_ref[...], preferred_element_type=jnp.float32
    )


def _mm(embedding, W):
    bm = 2000
    return pl.pallas_call(
        _mm_body,
        grid=(N_NODES // bm,),
        in_specs=[
            pl.BlockSpec((bm, D), lambda i: (i, 0)),
            pl.BlockSpec((D, D), lambda i: (0, 0)),
        ],
        out_specs=pl.BlockSpec((bm, D), lambda i: (i, 0)),
        out_shape=jax.ShapeDtypeStruct((N_NODES, D), jnp.float32),
    )(embedding, W)


def _scale_body(h_ref, deg_ref, out_ref, dis_ref):
    deg = deg_ref[0, :N_NODES] + deg_ref[1, :N_NODES] + 1.0
    dis = lax.rsqrt(deg)[:, None]
    out_ref[...] = h_ref[...] * dis
    dis_ref[...] = dis


def _scale(h, deg):
    return pl.pallas_call(
        _scale_body,
        out_shape=[
            jax.ShapeDtypeStruct((N_NODES, D), jnp.float32),
            jax.ShapeDtypeStruct((N_NODES, 1), jnp.float32),
        ],
    )(h, deg)


def _final_body(acc_ref, h2_ref, dis_ref, b_ref, out_ref):
    tot = acc_ref[0] + acc_ref[1] + h2_ref[...]
    out_ref[...] = tot * dis_ref[...] + b_ref[...]


def _final(acc, h2, dis, b2d):
    bm = 2000
    grid = N_NODES // bm
    return pl.pallas_call(
        _final_body,
        grid=(grid,),
        in_specs=[
            pl.BlockSpec((NC, bm, D), lambda i: (0, i, 0)),
            pl.BlockSpec((bm, D), lambda i: (i, 0)),
            pl.BlockSpec((bm, 1), lambda i: (i, 0)),
            pl.BlockSpec((1, D), lambda i: (0, 0)),
        ],
        out_specs=pl.BlockSpec((bm, D), lambda i: (i, 0)),
        out_shape=jax.ShapeDtypeStruct((N_NODES, D), jnp.float32),
    )(acc, h2, dis, b2d)


@jax.jit
def kernel(embedding, top_to_bottom_edge_index, W, b):
    edge_r = top_to_bottom_edge_index.astype(jnp.int32).reshape(
        2, NW, NCHUNK, CHUNK
    )
    h = _mm(embedding, W)
    deg = _deg_partials(edge_r)
    h2, dis = _scale(h, deg)
    acc = _msg_partials(h2, edge_r)
    return _final(acc, h2, dis, b.reshape(1, D))
